# baseline (speedup 1.0000x reference)
import functools

import jax
import jax.numpy as jnp
from jax import lax
from jax.experimental import pallas as pl
from jax.experimental.pallas import tpu as pltpu
from jax.experimental.pallas import tpu_sc as plsc


def _winner_body(x_ref, xsq_ref, wsq_ref, kwt_ref, win_ref):
    cross_m2 = jnp.dot(x_ref[...], kwt_ref[...], preferred_element_type=jnp.float32)
    dist = jnp.sqrt(jnp.maximum(xsq_ref[...] + wsq_ref[...] + cross_m2, 0.0))
    bt, h = cross_m2.shape
    dmin = jnp.min(dist, axis=1, keepdims=True)
    iota = lax.broadcasted_iota(jnp.int32, (bt, h), 1)
    win = jnp.min(jnp.where(dist == dmin, iota, h), axis=1).astype(jnp.int32)
    win_ref[...] = win[:, None]


def _winner_call(x, x_sq, w_sq, kwt_m2, bt, chunk_rows, chunk_idx):
    batch, in_dim = x.shape
    hidden = kwt_m2.shape[1]
    blk0 = chunk_idx * (chunk_rows // bt)
    return pl.pallas_call(
        _winner_body,
        grid=(chunk_rows // bt,),
        in_specs=[
            pl.BlockSpec((bt, in_dim), lambda i: (blk0 + i, 0)),
            pl.BlockSpec((bt, 1), lambda i: (blk0 + i, 0)),
            pl.BlockSpec((1, hidden), lambda i: (0, 0)),
            pl.BlockSpec((in_dim, hidden), lambda i: (0, 0)),
        ],
        out_specs=pl.BlockSpec((bt, 1), lambda i: (i, 0)),
        out_shape=jax.ShapeDtypeStruct((chunk_rows, 1), jnp.int32),
    )(x, x_sq, w_sq, kwt_m2)


def _make_sc_gather(batch, out_dim):
    info = plsc.get_sparse_core_info()
    nc, ns = info.num_cores, info.num_subcores
    nw = nc * ns
    b_per_w = batch // nw
    chunk = min(b_per_w, 256)
    n_chunks = b_per_w // chunk
    mesh = plsc.VectorSubcoreMesh(core_axis_name="c", subcore_axis_name="s")

    @functools.partial(
        pl.kernel, mesh=mesh,
        out_type=jax.ShapeDtypeStruct((batch, out_dim), jnp.float32),
        scratch_types=[
            pltpu.VMEM((chunk,), jnp.int32),
            pltpu.VMEM((chunk, out_dim), jnp.float32),
            pltpu.SemaphoreType.DMA,
        ],
    )
    def gather_kernel(table_hbm, idx_hbm, out_hbm, idx_v, rows_v, sem):
        wid = lax.axis_index("s") * nc + lax.axis_index("c")
        base = wid * b_per_w

        def body(i, _):
            off = base + i * chunk
            pltpu.sync_copy(idx_hbm.at[pl.ds(off, chunk)], idx_v)
            pltpu.async_copy(table_hbm.at[idx_v], rows_v, sem).wait()
            pltpu.sync_copy(rows_v, out_hbm.at[pl.ds(off, chunk)])
            return 0

        lax.fori_loop(0, n_chunks, body, 0)

    return gather_kernel


def kernel(x, kohonen_weights, grossberg_weights):
    batch, _ = x.shape
    out_dim = grossberg_weights.shape[0]
    x_sq = jnp.sum(x * x, axis=1, keepdims=True)
    w_sq = jnp.sum(kohonen_weights * kohonen_weights, axis=1)[None, :]
    kwt_m2 = kohonen_weights.T * -2.0
    gwt = grossberg_weights.T

    n_chunks = 2
    chunk_rows = batch // n_chunks
    gather = _make_sc_gather(chunk_rows, out_dim)
    wins, outs = [], []
    for c in range(n_chunks):
        win_c = _winner_call(x, x_sq, w_sq, kwt_m2, 1024, chunk_rows, c)[:, 0]
        wins.append(win_c)
        outs.append(gather(gwt, win_c))
    return (jnp.concatenate(outs, axis=0), jnp.concatenate(wins, axis=0))


# speedup vs baseline: 1.3955x; 1.3955x over previous
import jax
import jax.numpy as jnp
from jax import lax
from jax.experimental import pallas as pl


def _cpn_body(x_ref, xsq_ref, wsq_ref, kw_ref, gw_ref, out_ref, win_ref):
    cross_m2 = lax.dot_general(x_ref[...], kw_ref[...], (((1,), (1,)), ((), ())),
                               preferred_element_type=jnp.float32)
    dist = jnp.sqrt(jnp.maximum(xsq_ref[...] + wsq_ref[...] + cross_m2, 0.0))
    bt, h = dist.shape
    dmin = jnp.min(dist, axis=1, keepdims=True)
    iota = lax.broadcasted_iota(jnp.int32, (bt, h), 1)
    win = jnp.min(jnp.where(dist == dmin, iota, h), axis=1).astype(jnp.int32)
    win_ref[...] = win[:, None]
    one_hot = (iota == win[:, None]).astype(jnp.float32)
    out_ref[...] = lax.dot_general(one_hot, gw_ref[...], (((1,), (1,)), ((), ())),
                                   preferred_element_type=jnp.float32)


def kernel(x, kohonen_weights, grossberg_weights):
    batch, in_dim = x.shape
    hidden = kohonen_weights.shape[0]
    out_dim = grossberg_weights.shape[0]
    x_sq = jnp.sum(x * x, axis=1, keepdims=True)
    w_sq = jnp.sum(kohonen_weights * kohonen_weights, axis=1)[None, :]
    kw_m2 = kohonen_weights * -2.0

    bt = 1024
    out, win = pl.pallas_call(
        _cpn_body,
        grid=(batch // bt,),
        in_specs=[
            pl.BlockSpec((bt, in_dim), lambda i: (i, 0)),
            pl.BlockSpec((bt, 1), lambda i: (i, 0)),
            pl.BlockSpec((1, hidden), lambda i: (0, 0)),
            pl.BlockSpec((hidden, in_dim), lambda i: (0, 0)),
            pl.BlockSpec((out_dim, hidden), lambda i: (0, 0)),
        ],
        out_specs=[
            pl.BlockSpec((bt, out_dim), lambda i: (i, 0)),
            pl.BlockSpec((bt, 1), lambda i: (i, 0)),
        ],
        out_shape=[
            jax.ShapeDtypeStruct((batch, out_dim), jnp.float32),
            jax.ShapeDtypeStruct((batch, 1), jnp.int32),
        ],
    )(x, x_sq, w_sq, kw_m2, grossberg_weights)
    return (out, win[:, 0])


# speedup vs baseline: 1.4252x; 1.0213x over previous
import jax
import jax.numpy as jnp
from jax import lax
from jax.experimental import pallas as pl


def _cpn_body(x_ref, xsq_ref, wsq_ref, kw_ref, gw_ref, out_ref, win_ref):
    cross_m2 = lax.dot_general(x_ref[...], kw_ref[...], (((1,), (1,)), ((), ())),
                               preferred_element_type=jnp.float32)
    dist = jnp.sqrt(jnp.maximum(xsq_ref[...] + wsq_ref[...] + cross_m2, 0.0))
    bt, h = dist.shape
    dmin = jnp.min(dist, axis=1, keepdims=True)
    iota = lax.broadcasted_iota(jnp.int32, (bt, h), 1)
    win = jnp.min(jnp.where(dist == dmin, iota, h), axis=1).astype(jnp.int32)
    win_ref[...] = win[:, None]
    one_hot = (iota == win[:, None]).astype(jnp.float32)
    out_ref[...] = lax.dot_general(one_hot, gw_ref[...], (((1,), (1,)), ((), ())),
                                   preferred_element_type=jnp.float32)


def kernel(x, kohonen_weights, grossberg_weights):
    batch, in_dim = x.shape
    hidden = kohonen_weights.shape[0]
    out_dim = grossberg_weights.shape[0]
    x_sq = jnp.sum(x * x, axis=1, keepdims=True)
    w_sq = jnp.sum(kohonen_weights * kohonen_weights, axis=1)[None, :]
    kw_m2 = kohonen_weights * -2.0

    bt = 2048
    out, win = pl.pallas_call(
        _cpn_body,
        grid=(batch // bt,),
        in_specs=[
            pl.BlockSpec((bt, in_dim), lambda i: (i, 0)),
            pl.BlockSpec((bt, 1), lambda i: (i, 0)),
            pl.BlockSpec((1, hidden), lambda i: (0, 0)),
            pl.BlockSpec((hidden, in_dim), lambda i: (0, 0)),
            pl.BlockSpec((out_dim, hidden), lambda i: (0, 0)),
        ],
        out_specs=[
            pl.BlockSpec((bt, out_dim), lambda i: (i, 0)),
            pl.BlockSpec((bt, 1), lambda i: (i, 0)),
        ],
        out_shape=[
            jax.ShapeDtypeStruct((batch, out_dim), jnp.float32),
            jax.ShapeDtypeStruct((batch, 1), jnp.int32),
        ],
    )(x, x_sq, w_sq, kw_m2, grossberg_weights)
    return (out, win[:, 0])
